# Initial kernel scaffold; baseline (speedup 1.0000x reference)
import jax
import jax.numpy as jnp
from jax.experimental import pallas as pl


def kernel(x, weight):
    raise NotImplementedError("write your pallas kernel here")


# speedup vs baseline: 1.9177x; 1.9177x over previous
import jax
import jax.numpy as jnp
from jax.experimental import pallas as pl
from jax.experimental.pallas import tpu as pltpu

NUM_EXPERTS = 64
TOP_K = 2
ALPHA = 0.01
DIM = 2048
T = 16384

BLK = 1024


def _gate_kernel(x_ref, wt_ref, idx_ref, w_ref, aux_ref, acc_ref):
    i = pl.program_id(0)
    n = pl.num_programs(0)

    @pl.when(i == 0)
    def _init():
        acc_ref[...] = jnp.zeros_like(acc_ref)

    logits = jnp.dot(x_ref[...], wt_ref[...],
                     preferred_element_type=jnp.float32)

    m = jnp.max(logits, axis=-1, keepdims=True)
    e = jnp.exp(logits - m)
    s = jnp.sum(e, axis=-1, keepdims=True)
    scores = e / s

    col = jax.lax.broadcasted_iota(jnp.int32, logits.shape, 1)
    is1 = logits == m
    idx1 = jnp.min(jnp.where(is1, col, NUM_EXPERTS), axis=-1, keepdims=True)
    masked = jnp.where(col == idx1, -jnp.inf, logits)
    m2 = jnp.max(masked, axis=-1, keepdims=True)
    is2 = masked == m2
    idx2 = jnp.min(jnp.where(is2, col, NUM_EXPERTS), axis=-1, keepdims=True)

    s1 = jnp.sum(jnp.where(col == idx1, scores, 0.0), axis=-1, keepdims=True)
    s2 = jnp.sum(jnp.where(col == idx2, scores, 0.0), axis=-1, keepdims=True)
    tot = s1 + s2

    idx_ref[...] = jnp.concatenate([idx1, idx2], axis=1)
    w_ref[...] = jnp.concatenate([s1 / tot, s2 / tot], axis=1)

    pi_part = jnp.sum(scores, axis=0, keepdims=True)
    cnt_part = (jnp.sum((col == idx1).astype(jnp.float32), axis=0, keepdims=True)
                + jnp.sum((col == idx2).astype(jnp.float32), axis=0, keepdims=True))
    acc_ref[0:1, :] += pi_part
    acc_ref[1:2, :] += cnt_part

    @pl.when(i == n - 1)
    def _fin():
        pi = acc_ref[0:1, :] / jnp.float32(T)
        fi = acc_ref[1:2, :] * jnp.float32(NUM_EXPERTS / (T * TOP_K))
        aux_ref[...] = (jnp.sum(pi * fi) * jnp.float32(ALPHA)).reshape(1, 1)


def kernel(x, weight):
    wt = weight.astype(jnp.float32).T
    grid = (T // BLK,)
    idx, w, aux = pl.pallas_call(
        _gate_kernel,
        grid=grid,
        in_specs=[
            pl.BlockSpec((BLK, DIM), lambda i: (i, 0)),
            pl.BlockSpec((DIM, NUM_EXPERTS), lambda i: (0, 0)),
        ],
        out_specs=[
            pl.BlockSpec((BLK, TOP_K), lambda i: (i, 0)),
            pl.BlockSpec((BLK, TOP_K), lambda i: (i, 0)),
            pl.BlockSpec((1, 1), lambda i: (0, 0)),
        ],
        out_shape=[
            jax.ShapeDtypeStruct((T, TOP_K), jnp.int32),
            jax.ShapeDtypeStruct((T, TOP_K), jnp.float32),
            jax.ShapeDtypeStruct((1, 1), jnp.float32),
        ],
        scratch_shapes=[pltpu.VMEM((2, NUM_EXPERTS), jnp.float32)],
        compiler_params=pltpu.CompilerParams(
            dimension_semantics=("arbitrary",),
        ),
    )(x.astype(jnp.float32), wt)
    return (idx, w, aux.reshape(()))


# speedup vs baseline: 2.0414x; 1.0645x over previous
import jax
import jax.numpy as jnp
from jax.experimental import pallas as pl
from jax.experimental.pallas import tpu as pltpu

NUM_EXPERTS = 64
TOP_K = 2
ALPHA = 0.01
DIM = 2048
T = 16384

BLK = 1024


def _gate_kernel(x_ref, wt_ref, idx_ref, w_ref, aux_ref, acc_ref):
    i = pl.program_id(0)
    n = pl.num_programs(0)

    @pl.when(i == 0)
    def _init():
        acc_ref[...] = jnp.zeros_like(acc_ref)

    logits = jnp.dot(x_ref[...], wt_ref[...],
                     preferred_element_type=jnp.float32)

    col = jax.lax.broadcasted_iota(jnp.int32, logits.shape, 1)
    colrev = ((NUM_EXPERTS - 1) - col).astype(jnp.float32)

    m1 = jnp.max(logits, axis=-1, keepdims=True)
    c1 = logits == m1
    a1 = jnp.max(jnp.where(c1, colrev, -1.0), axis=-1, keepdims=True)
    idx1 = (jnp.float32(NUM_EXPERTS - 1) - a1).astype(jnp.int32)
    masked = jnp.where(c1, -jnp.inf, logits)
    m2 = jnp.max(masked, axis=-1, keepdims=True)
    c2 = masked == m2
    a2 = jnp.max(jnp.where(c2, colrev, -1.0), axis=-1, keepdims=True)
    idx2 = (jnp.float32(NUM_EXPERTS - 1) - a2).astype(jnp.int32)

    r = jnp.exp(m2 - m1)
    w1 = 1.0 / (1.0 + r)
    w2 = 1.0 - w1

    idx_ref[...] = jnp.concatenate([idx1, idx2], axis=1)
    w_ref[...] = jnp.concatenate([w1, w2], axis=1)

    e = jnp.exp(logits - m1)
    s = jnp.sum(e, axis=-1, keepdims=True)
    scores = e * (1.0 / s)

    pi_part = jnp.sum(scores, axis=0, keepdims=True)
    cnt_part = jnp.sum(c1.astype(jnp.float32) + c2.astype(jnp.float32),
                       axis=0, keepdims=True)
    acc_ref[0:1, :] += pi_part
    acc_ref[1:2, :] += cnt_part

    @pl.when(i == n - 1)
    def _fin():
        pi = acc_ref[0:1, :] / jnp.float32(T)
        fi = acc_ref[1:2, :] * jnp.float32(NUM_EXPERTS / (T * TOP_K))
        aux_ref[...] = (jnp.sum(pi * fi) * jnp.float32(ALPHA)).reshape(1, 1)


def kernel(x, weight):
    wt = weight.astype(jnp.float32).T
    grid = (T // BLK,)
    idx, w, aux = pl.pallas_call(
        _gate_kernel,
        grid=grid,
        in_specs=[
            pl.BlockSpec((BLK, DIM), lambda i: (i, 0)),
            pl.BlockSpec((DIM, NUM_EXPERTS), lambda i: (0, 0)),
        ],
        out_specs=[
            pl.BlockSpec((BLK, TOP_K), lambda i: (i, 0)),
            pl.BlockSpec((BLK, TOP_K), lambda i: (i, 0)),
            pl.BlockSpec((1, 1), lambda i: (0, 0)),
        ],
        out_shape=[
            jax.ShapeDtypeStruct((T, TOP_K), jnp.int32),
            jax.ShapeDtypeStruct((T, TOP_K), jnp.float32),
            jax.ShapeDtypeStruct((1, 1), jnp.float32),
        ],
        scratch_shapes=[pltpu.VMEM((2, NUM_EXPERTS), jnp.float32)],
        compiler_params=pltpu.CompilerParams(
            dimension_semantics=("arbitrary",),
        ),
    )(x.astype(jnp.float32), wt)
    return (idx, w, aux.reshape(()))


# speedup vs baseline: 2.0966x; 1.0270x over previous
import jax
import jax.numpy as jnp
from jax.experimental import pallas as pl
from jax.experimental.pallas import tpu as pltpu

NUM_EXPERTS = 64
TOP_K = 2
ALPHA = 0.01
DIM = 2048
T = 16384

BLK = 2048


def _gate_kernel(x_ref, wt_ref, idx_ref, w_ref, aux_ref, acc_ref):
    i = pl.program_id(0)
    n = pl.num_programs(0)

    @pl.when(i == 0)
    def _init():
        acc_ref[...] = jnp.zeros_like(acc_ref)

    logits = jnp.dot(x_ref[...], wt_ref[...],
                     preferred_element_type=jnp.float32)

    col = jax.lax.broadcasted_iota(jnp.int32, logits.shape, 1)
    colrev = ((NUM_EXPERTS - 1) - col).astype(jnp.float32)

    m1 = jnp.max(logits, axis=-1, keepdims=True)
    c1 = logits == m1
    a1 = jnp.max(jnp.where(c1, colrev, -1.0), axis=-1, keepdims=True)
    idx1 = (jnp.float32(NUM_EXPERTS - 1) - a1).astype(jnp.int32)
    masked = jnp.where(c1, -jnp.inf, logits)
    m2 = jnp.max(masked, axis=-1, keepdims=True)
    c2 = masked == m2
    a2 = jnp.max(jnp.where(c2, colrev, -1.0), axis=-1, keepdims=True)
    idx2 = (jnp.float32(NUM_EXPERTS - 1) - a2).astype(jnp.int32)

    r = jnp.exp(m2 - m1)
    w1 = 1.0 / (1.0 + r)
    w2 = 1.0 - w1

    idx_ref[...] = jnp.concatenate([idx1, idx2], axis=1)
    w_ref[...] = jnp.concatenate([w1, w2], axis=1)

    e = jnp.exp(logits - m1)
    s = jnp.sum(e, axis=-1, keepdims=True)
    scores = e * (1.0 / s)

    pi_part = jnp.sum(scores, axis=0, keepdims=True)
    cnt_part = jnp.sum(c1.astype(jnp.float32) + c2.astype(jnp.float32),
                       axis=0, keepdims=True)
    acc_ref[0:1, :] += pi_part
    acc_ref[1:2, :] += cnt_part

    @pl.when(i == n - 1)
    def _fin():
        pi = acc_ref[0:1, :] / jnp.float32(T)
        fi = acc_ref[1:2, :] * jnp.float32(NUM_EXPERTS / (T * TOP_K))
        aux_ref[...] = (jnp.sum(pi * fi) * jnp.float32(ALPHA)).reshape(1, 1)


def kernel(x, weight):
    wt = weight.astype(jnp.float32).T
    grid = (T // BLK,)
    idx, w, aux = pl.pallas_call(
        _gate_kernel,
        grid=grid,
        in_specs=[
            pl.BlockSpec((BLK, DIM), lambda i: (i, 0)),
            pl.BlockSpec((DIM, NUM_EXPERTS), lambda i: (0, 0)),
        ],
        out_specs=[
            pl.BlockSpec((BLK, TOP_K), lambda i: (i, 0)),
            pl.BlockSpec((BLK, TOP_K), lambda i: (i, 0)),
            pl.BlockSpec((1, 1), lambda i: (0, 0)),
        ],
        out_shape=[
            jax.ShapeDtypeStruct((T, TOP_K), jnp.int32),
            jax.ShapeDtypeStruct((T, TOP_K), jnp.float32),
            jax.ShapeDtypeStruct((1, 1), jnp.float32),
        ],
        scratch_shapes=[pltpu.VMEM((2, NUM_EXPERTS), jnp.float32)],
        compiler_params=pltpu.CompilerParams(
            dimension_semantics=("arbitrary",),
        ),
    )(x.astype(jnp.float32), wt)
    return (idx, w, aux.reshape(()))
